# Initial kernel scaffold; baseline (speedup 1.0000x reference)
import jax
import jax.numpy as jnp
from jax.experimental import pallas as pl


def kernel(x, edge_index, W_gcn, b_gcn, W_proj, b_proj):
    raise NotImplementedError("write your pallas kernel here")


# speedup vs baseline: 5.9724x; 5.9724x over previous
import functools

import jax
import jax.numpy as jnp
from jax.experimental import pallas as pl

_N_NODES = 102
_IN = 2
_HID = 512
_EMB = 256
def _agg_kernel(x2_ref, rc_ref, y_ref):
    rc = rc_ref[...]
    row = rc[0:1, :]
    col = rc[1:2, :]
    e_tot = rc.shape[1]
    iota = jax.lax.broadcasted_iota(jnp.int32, (_N_NODES, e_tot), 0)
    cmask = (iota == col).astype(jnp.float32)
    rmask = (iota == row).astype(jnp.float32)
    deg = jnp.sum(cmask, axis=1, keepdims=True)
    dinv = jnp.where(deg > 0, jax.lax.rsqrt(deg), 0.0)
    dinv_col = jnp.sum(cmask * dinv, axis=0, keepdims=True)
    dinv_row = jnp.sum(rmask * dinv, axis=0, keepdims=True)
    norm = dinv_row * dinv_col
    a_t = jax.lax.dot_general(rmask * norm, cmask,
                              (((1,), (1,)), ((), ())),
                              preferred_element_type=jnp.float32)
    y_ref[...] = jnp.dot(x2_ref[...], a_t, preferred_element_type=jnp.float32)


def _dense_kernel(z_ref, wg_ref, bg_ref, wp_ref, bp_ref, o_ref):
    z = z_ref[...]
    h = (z[:, 0:1] * wg_ref[0:1, :] + z[:, 1:2] * wg_ref[1:2, :]
         + bg_ref[...])
    h = jnp.maximum(h, 0.0)
    o_ref[...] = (jnp.dot(h, wp_ref[...], preferred_element_type=jnp.float32)
                  + bp_ref[...])


@functools.partial(jax.jit, static_argnames=())
def kernel(x, edge_index, W_gcn, b_gcn, W_proj, b_proj):
    B, n, _ = x.shape
    F = B * n
    xr = x.reshape(F, _N_NODES, _IN)
    x2 = jnp.concatenate([xr[:, :, 0], xr[:, :, 1]], axis=0)

    loops = jnp.arange(_N_NODES, dtype=edge_index.dtype)
    rc = jnp.concatenate(
        [edge_index, jnp.stack([loops, loops], axis=0)], axis=1)

    y = pl.pallas_call(
        _agg_kernel,
        out_shape=jax.ShapeDtypeStruct((2 * F, _N_NODES), jnp.float32),
    )(x2, rc)

    z = jnp.stack([y[:F], y[F:]], axis=-1).reshape(F * _N_NODES, _IN)

    rows = F * _N_NODES
    tile = 3264
    grid = rows // tile

    out_flat = pl.pallas_call(
        _dense_kernel,
        grid=(grid,),
        in_specs=[
            pl.BlockSpec((tile, _IN), lambda i: (i, 0)),
            pl.BlockSpec((_IN, _HID), lambda i: (0, 0)),
            pl.BlockSpec((1, _HID), lambda i: (0, 0)),
            pl.BlockSpec((_HID, _EMB), lambda i: (0, 0)),
            pl.BlockSpec((1, _EMB), lambda i: (0, 0)),
        ],
        out_specs=pl.BlockSpec((tile, _EMB), lambda i: (i, 0)),
        out_shape=jax.ShapeDtypeStruct((rows, _EMB), jnp.float32),
    )(z, W_gcn, b_gcn.reshape(1, _HID), W_proj, b_proj.reshape(1, _EMB))

    return out_flat.reshape(B, n, _N_NODES, _EMB)


# speedup vs baseline: 6.2702x; 1.0499x over previous
import functools

import jax
import jax.numpy as jnp
from jax.experimental import pallas as pl
from jax.experimental.pallas import tpu as pltpu

_N_NODES = 102
_IN = 2
_HID = 512
_EMB = 256
_TILE_F = 32


def _fused_kernel(x_ref, rc_ref, wg_ref, bg_ref, wp_ref, bp_ref, o_ref,
                  me_ref, mo_ref):
    t = x_ref.shape[0]

    @pl.when(pl.program_id(0) == 0)
    def _build_adjacency():
        rc = rc_ref[...]
        row = rc[0:1, :]
        col = rc[1:2, :]
        e_tot = rc.shape[1]
        iota_n = jax.lax.broadcasted_iota(jnp.int32, (_N_NODES, e_tot), 0)
        cmask = (iota_n == col).astype(jnp.float32)
        rmask = (iota_n == row).astype(jnp.float32)
        deg = jnp.sum(cmask, axis=1, keepdims=True)
        dinv = jnp.where(deg > 0, jax.lax.rsqrt(deg), 0.0)
        dinv_col = jnp.sum(cmask * dinv, axis=0, keepdims=True)
        dinv_row = jnp.sum(rmask * dinv, axis=0, keepdims=True)
        norm = dinv_row * dinv_col
        iota_k = jax.lax.broadcasted_iota(jnp.int32, (2 * _N_NODES, e_tot), 0)
        pe = (iota_k == 2 * row).astype(jnp.float32)
        po = (iota_k == 2 * row + 1).astype(jnp.float32)
        q = cmask * norm
        dims = (((1,), (1,)), ((), ()))
        me_ref[...] = jax.lax.dot_general(pe, q, dims,
                                          preferred_element_type=jnp.float32)
        mo_ref[...] = jax.lax.dot_general(po, q, dims,
                                          preferred_element_type=jnp.float32)

    x = x_ref[...]
    ye = jnp.dot(x, me_ref[...], preferred_element_type=jnp.float32)
    yo = jnp.dot(x, mo_ref[...], preferred_element_type=jnp.float32)
    wg = wg_ref[...]
    h = (ye[:, :, None] * wg[0][None, None, :]
         + yo[:, :, None] * wg[1][None, None, :]
         + bg_ref[...][None])
    h = jnp.maximum(h, 0.0).reshape(t * _N_NODES, _HID)
    o_ref[...] = (jnp.dot(h, wp_ref[...], preferred_element_type=jnp.float32)
                  + bp_ref[...])


@functools.partial(jax.jit, static_argnames=())
def kernel(x, edge_index, W_gcn, b_gcn, W_proj, b_proj):
    B, n, _ = x.shape
    F = B * n
    x2 = x.reshape(F, _IN * _N_NODES)

    loops = jnp.arange(_N_NODES, dtype=edge_index.dtype)
    rc = jnp.concatenate(
        [edge_index, jnp.stack([loops, loops], axis=0)], axis=1)

    grid = F // _TILE_F
    rows = F * _N_NODES

    out_flat = pl.pallas_call(
        _fused_kernel,
        grid=(grid,),
        in_specs=[
            pl.BlockSpec((_TILE_F, _IN * _N_NODES), lambda i: (i, 0)),
            pl.BlockSpec(rc.shape, lambda i: (0, 0)),
            pl.BlockSpec((_IN, _HID), lambda i: (0, 0)),
            pl.BlockSpec((1, _HID), lambda i: (0, 0)),
            pl.BlockSpec((_HID, _EMB), lambda i: (0, 0)),
            pl.BlockSpec((1, _EMB), lambda i: (0, 0)),
        ],
        out_specs=pl.BlockSpec((_TILE_F * _N_NODES, _EMB), lambda i: (i, 0)),
        out_shape=jax.ShapeDtypeStruct((rows, _EMB), jnp.float32),
        scratch_shapes=[
            pltpu.VMEM((_IN * _N_NODES, _N_NODES), jnp.float32),
            pltpu.VMEM((_IN * _N_NODES, _N_NODES), jnp.float32),
        ],
    )(x2, rc, W_gcn, b_gcn.reshape(1, _HID), W_proj, b_proj.reshape(1, _EMB))

    return out_flat.reshape(B, n, _N_NODES, _EMB)


# speedup vs baseline: 11.3894x; 1.8164x over previous
import functools

import jax
import jax.numpy as jnp
from jax.experimental import pallas as pl
from jax.experimental.pallas import tpu as pltpu

_N_NODES = 102
_NPAD = 104
_IN = 2
_HID = 512
_EMB = 256
_TILE_F = 32


def _fused_kernel(x_ref, rc_ref, wg_ref, bg_ref, wp_ref, bp_ref, o_ref,
                  me_ref, mo_ref):
    t = x_ref.shape[0]

    @pl.when(pl.program_id(0) == 0)
    def _build_adjacency():
        rc = rc_ref[...]
        row = rc[0:1, :]
        col = rc[1:2, :]
        e_tot = rc.shape[1]
        iota_n = jax.lax.broadcasted_iota(jnp.int32, (_NPAD, e_tot), 0)
        cmask = (iota_n == col).astype(jnp.float32)
        rmask = (iota_n == row).astype(jnp.float32)
        deg = jnp.sum(cmask, axis=1, keepdims=True)
        dinv = jnp.where(deg > 0, jax.lax.rsqrt(deg), 0.0)
        dinv_col = jnp.sum(cmask * dinv, axis=0, keepdims=True)
        dinv_row = jnp.sum(rmask * dinv, axis=0, keepdims=True)
        norm = dinv_row * dinv_col
        iota_k = jax.lax.broadcasted_iota(jnp.int32, (2 * _N_NODES, e_tot), 0)
        pe = (iota_k == 2 * row).astype(jnp.float32)
        po = (iota_k == 2 * row + 1).astype(jnp.float32)
        q = cmask * norm
        dims = (((1,), (1,)), ((), ()))
        me_ref[...] = jax.lax.dot_general(pe, q, dims,
                                          preferred_element_type=jnp.float32)
        mo_ref[...] = jax.lax.dot_general(po, q, dims,
                                          preferred_element_type=jnp.float32)

    x = x_ref[...]
    ye = jnp.dot(x, me_ref[...], preferred_element_type=jnp.float32)
    yo = jnp.dot(x, mo_ref[...], preferred_element_type=jnp.float32)
    wg = wg_ref[...]
    h = (ye[:, :, None] * wg[0][None, None, :]
         + yo[:, :, None] * wg[1][None, None, :]
         + bg_ref[...][None])
    h = jnp.maximum(h, 0.0).reshape(t * _NPAD, _HID)
    o2 = (jnp.dot(h, wp_ref[...], preferred_element_type=jnp.float32)
          + bp_ref[...])
    for f in range(t):
        o_ref[f] = o2[f * _NPAD:f * _NPAD + _N_NODES, :]


@functools.partial(jax.jit, static_argnames=())
def kernel(x, edge_index, W_gcn, b_gcn, W_proj, b_proj):
    B, n, _ = x.shape
    F = B * n
    x2 = x.reshape(F, _IN * _N_NODES)

    loops = jnp.arange(_N_NODES, dtype=edge_index.dtype)
    rc = jnp.concatenate(
        [edge_index, jnp.stack([loops, loops], axis=0)], axis=1)

    grid = F // _TILE_F

    out4 = pl.pallas_call(
        _fused_kernel,
        grid=(grid,),
        in_specs=[
            pl.BlockSpec((_TILE_F, _IN * _N_NODES), lambda i: (i, 0)),
            pl.BlockSpec(rc.shape, lambda i: (0, 0)),
            pl.BlockSpec((_IN, _HID), lambda i: (0, 0)),
            pl.BlockSpec((1, _HID), lambda i: (0, 0)),
            pl.BlockSpec((_HID, _EMB), lambda i: (0, 0)),
            pl.BlockSpec((1, _EMB), lambda i: (0, 0)),
        ],
        out_specs=pl.BlockSpec((_TILE_F, _N_NODES, _EMB), lambda i: (i, 0, 0)),
        out_shape=jax.ShapeDtypeStruct((F, _N_NODES, _EMB), jnp.float32),
        scratch_shapes=[
            pltpu.VMEM((_IN * _N_NODES, _NPAD), jnp.float32),
            pltpu.VMEM((_IN * _N_NODES, _NPAD), jnp.float32),
        ],
    )(x2, rc, W_gcn, b_gcn.reshape(1, _HID), W_proj, b_proj.reshape(1, _EMB))

    return out4.reshape(B, n, _N_NODES, _EMB)


# speedup vs baseline: 11.7324x; 1.0301x over previous
import functools

import jax
import jax.numpy as jnp
from jax.experimental import pallas as pl
from jax.experimental.pallas import tpu as pltpu

_N_NODES = 102
_NPAD = 104
_IN = 2
_HID = 512
_EMB = 256
_TILE_F = 64


def _fused_kernel(x_ref, rc_ref, wg_ref, bg_ref, wp_ref, bp_ref, o_ref,
                  me_ref, mo_ref):
    t = x_ref.shape[0]

    @pl.when(pl.program_id(0) == 0)
    def _build_adjacency():
        rc = rc_ref[...]
        row = rc[0:1, :]
        col = rc[1:2, :]
        e_tot = rc.shape[1]
        iota_n = jax.lax.broadcasted_iota(jnp.int32, (_NPAD, e_tot), 0)
        cmask = (iota_n == col).astype(jnp.float32)
        rmask = (iota_n == row).astype(jnp.float32)
        deg = jnp.sum(cmask, axis=1, keepdims=True)
        dinv = jnp.where(deg > 0, jax.lax.rsqrt(deg), 0.0)
        dinv_col = jnp.sum(cmask * dinv, axis=0, keepdims=True)
        dinv_row = jnp.sum(rmask * dinv, axis=0, keepdims=True)
        norm = dinv_row * dinv_col
        iota_k = jax.lax.broadcasted_iota(jnp.int32, (2 * _N_NODES, e_tot), 0)
        pe = (iota_k == 2 * row).astype(jnp.float32)
        po = (iota_k == 2 * row + 1).astype(jnp.float32)
        q = cmask * norm
        dims = (((1,), (1,)), ((), ()))
        me_ref[...] = jax.lax.dot_general(pe, q, dims,
                                          preferred_element_type=jnp.float32)
        mo_ref[...] = jax.lax.dot_general(po, q, dims,
                                          preferred_element_type=jnp.float32)

    x = x_ref[...]
    ye = jnp.dot(x, me_ref[...], preferred_element_type=jnp.float32)
    yo = jnp.dot(x, mo_ref[...], preferred_element_type=jnp.float32)
    wg = wg_ref[...]
    h = (ye[:, :, None] * wg[0][None, None, :]
         + yo[:, :, None] * wg[1][None, None, :]
         + bg_ref[...][None])
    h = jnp.maximum(h, 0.0).reshape(t * _NPAD, _HID)
    o2 = (jnp.dot(h, wp_ref[...], preferred_element_type=jnp.float32)
          + bp_ref[...])
    for f in range(t):
        o_ref[f] = o2[f * _NPAD:f * _NPAD + _N_NODES, :]


@functools.partial(jax.jit, static_argnames=())
def kernel(x, edge_index, W_gcn, b_gcn, W_proj, b_proj):
    B, n, _ = x.shape
    F = B * n
    x2 = x.reshape(F, _IN * _N_NODES)

    loops = jnp.arange(_N_NODES, dtype=edge_index.dtype)
    rc = jnp.concatenate(
        [edge_index, jnp.stack([loops, loops], axis=0)], axis=1)

    grid = F // _TILE_F

    out4 = pl.pallas_call(
        _fused_kernel,
        grid=(grid,),
        in_specs=[
            pl.BlockSpec((_TILE_F, _IN * _N_NODES), lambda i: (i, 0)),
            pl.BlockSpec(rc.shape, lambda i: (0, 0)),
            pl.BlockSpec((_IN, _HID), lambda i: (0, 0)),
            pl.BlockSpec((1, _HID), lambda i: (0, 0)),
            pl.BlockSpec((_HID, _EMB), lambda i: (0, 0)),
            pl.BlockSpec((1, _EMB), lambda i: (0, 0)),
        ],
        out_specs=pl.BlockSpec((_TILE_F, _N_NODES, _EMB), lambda i: (i, 0, 0)),
        out_shape=jax.ShapeDtypeStruct((F, _N_NODES, _EMB), jnp.float32),
        scratch_shapes=[
            pltpu.VMEM((_IN * _N_NODES, _NPAD), jnp.float32),
            pltpu.VMEM((_IN * _N_NODES, _NPAD), jnp.float32),
        ],
    )(x2, rc, W_gcn, b_gcn.reshape(1, _HID), W_proj, b_proj.reshape(1, _EMB))

    return out4.reshape(B, n, _N_NODES, _EMB)


# speedup vs baseline: 11.7999x; 1.0058x over previous
import functools

import jax
import jax.numpy as jnp
from jax.experimental import pallas as pl
from jax.experimental.pallas import tpu as pltpu

_N_NODES = 102
_NPAD = 104
_IN = 2
_HID = 512
_EMB = 256
_TILE_F = 128


def _fused_kernel(x_ref, rc_ref, wg_ref, bg_ref, wp_ref, bp_ref, o_ref,
                  me_ref, mo_ref):
    t = x_ref.shape[0]

    @pl.when(pl.program_id(0) == 0)
    def _build_adjacency():
        rc = rc_ref[...]
        row = rc[0:1, :]
        col = rc[1:2, :]
        e_tot = rc.shape[1]
        iota_n = jax.lax.broadcasted_iota(jnp.int32, (_NPAD, e_tot), 0)
        cmask = (iota_n == col).astype(jnp.float32)
        rmask = (iota_n == row).astype(jnp.float32)
        deg = jnp.sum(cmask, axis=1, keepdims=True)
        dinv = jnp.where(deg > 0, jax.lax.rsqrt(deg), 0.0)
        dinv_col = jnp.sum(cmask * dinv, axis=0, keepdims=True)
        dinv_row = jnp.sum(rmask * dinv, axis=0, keepdims=True)
        norm = dinv_row * dinv_col
        iota_k = jax.lax.broadcasted_iota(jnp.int32, (2 * _N_NODES, e_tot), 0)
        pe = (iota_k == 2 * row).astype(jnp.float32)
        po = (iota_k == 2 * row + 1).astype(jnp.float32)
        q = cmask * norm
        dims = (((1,), (1,)), ((), ()))
        me_ref[...] = jax.lax.dot_general(pe, q, dims,
                                          preferred_element_type=jnp.float32)
        mo_ref[...] = jax.lax.dot_general(po, q, dims,
                                          preferred_element_type=jnp.float32)

    x = x_ref[...]
    ye = jnp.dot(x, me_ref[...], preferred_element_type=jnp.float32)
    yo = jnp.dot(x, mo_ref[...], preferred_element_type=jnp.float32)
    wg = wg_ref[...]
    h = (ye[:, :, None] * wg[0][None, None, :]
         + yo[:, :, None] * wg[1][None, None, :]
         + bg_ref[...][None])
    h = jnp.maximum(h, 0.0).reshape(t * _NPAD, _HID)
    o2 = (jnp.dot(h, wp_ref[...], preferred_element_type=jnp.float32)
          + bp_ref[...])
    for f in range(t):
        o_ref[f] = o2[f * _NPAD:f * _NPAD + _N_NODES, :]


@functools.partial(jax.jit, static_argnames=())
def kernel(x, edge_index, W_gcn, b_gcn, W_proj, b_proj):
    B, n, _ = x.shape
    F = B * n
    x2 = x.reshape(F, _IN * _N_NODES)

    loops = jnp.arange(_N_NODES, dtype=edge_index.dtype)
    rc = jnp.concatenate(
        [edge_index, jnp.stack([loops, loops], axis=0)], axis=1)

    grid = F // _TILE_F

    out4 = pl.pallas_call(
        _fused_kernel,
        grid=(grid,),
        in_specs=[
            pl.BlockSpec((_TILE_F, _IN * _N_NODES), lambda i: (i, 0)),
            pl.BlockSpec(rc.shape, lambda i: (0, 0)),
            pl.BlockSpec((_IN, _HID), lambda i: (0, 0)),
            pl.BlockSpec((1, _HID), lambda i: (0, 0)),
            pl.BlockSpec((_HID, _EMB), lambda i: (0, 0)),
            pl.BlockSpec((1, _EMB), lambda i: (0, 0)),
        ],
        out_specs=pl.BlockSpec((_TILE_F, _N_NODES, _EMB), lambda i: (i, 0, 0)),
        out_shape=jax.ShapeDtypeStruct((F, _N_NODES, _EMB), jnp.float32),
        scratch_shapes=[
            pltpu.VMEM((_IN * _N_NODES, _NPAD), jnp.float32),
            pltpu.VMEM((_IN * _N_NODES, _NPAD), jnp.float32),
        ],
    )(x2, rc, W_gcn, b_gcn.reshape(1, _HID), W_proj, b_proj.reshape(1, _EMB))

    return out4.reshape(B, n, _N_NODES, _EMB)


# speedup vs baseline: 26.0416x; 2.2069x over previous
import functools

import jax
import jax.numpy as jnp
from jax.experimental import pallas as pl
from jax.experimental.pallas import tpu as pltpu

_N_NODES = 102
_IN = 2
_HID = 512
_EMB = 256
_FCHUNK = 32


def _fused_kernel(x_ref, rc_ref, wg_ref, bg_ref, wp_ref, bp_ref, o_ref,
                  me_ref, mo_ref):
    fc = x_ref.shape[1]

    @pl.when((pl.program_id(0) == 0) & (pl.program_id(1) == 0))
    def _build_adjacency():
        rc = rc_ref[...]
        row = rc[0:1, :]
        col = rc[1:2, :]
        e_tot = rc.shape[1]
        iota_n = jax.lax.broadcasted_iota(jnp.int32, (_N_NODES, e_tot), 0)
        cmask = (iota_n == col).astype(jnp.float32)
        rmask = (iota_n == row).astype(jnp.float32)
        deg = jnp.sum(cmask, axis=1, keepdims=True)
        dinv = jnp.where(deg > 0, jax.lax.rsqrt(deg), 0.0)
        dinv_col = jnp.sum(cmask * dinv, axis=0, keepdims=True)
        dinv_row = jnp.sum(rmask * dinv, axis=0, keepdims=True)
        norm = dinv_row * dinv_col
        iota_k = jax.lax.broadcasted_iota(jnp.int32, (2 * _N_NODES, e_tot), 0)
        pe = (iota_k == 2 * row).astype(jnp.float32)
        po = (iota_k == 2 * row + 1).astype(jnp.float32)
        q = cmask * norm
        dims = (((1,), (1,)), ((), ()))
        me_ref[...] = jax.lax.dot_general(pe, q, dims,
                                          preferred_element_type=jnp.float32)
        mo_ref[...] = jax.lax.dot_general(po, q, dims,
                                          preferred_element_type=jnp.float32)

    x = x_ref[0]
    cdims = (((0,), (1,)), ((), ()))
    yet = jax.lax.dot_general(me_ref[...], x, cdims,
                              preferred_element_type=jnp.float32)
    yot = jax.lax.dot_general(mo_ref[...], x, cdims,
                              preferred_element_type=jnp.float32)
    wg = wg_ref[...]
    h = (yet[:, :, None] * wg[0][None, None, :]
         + yot[:, :, None] * wg[1][None, None, :]
         + bg_ref[...][None])
    h = jnp.maximum(h, 0.0).reshape(_N_NODES * fc, _HID)
    o2 = (jnp.dot(h, wp_ref[...], preferred_element_type=jnp.float32)
          + bp_ref[...])
    o_ref[0] = o2.reshape(_N_NODES, fc, _EMB)


@functools.partial(jax.jit, static_argnames=())
def kernel(x, edge_index, W_gcn, b_gcn, W_proj, b_proj):
    B, n, _ = x.shape

    loops = jnp.arange(_N_NODES, dtype=edge_index.dtype)
    rc = jnp.concatenate(
        [edge_index, jnp.stack([loops, loops], axis=0)], axis=1)

    grid = (B, n // _FCHUNK)

    out_nm = pl.pallas_call(
        _fused_kernel,
        grid=grid,
        in_specs=[
            pl.BlockSpec((1, _FCHUNK, _IN * _N_NODES), lambda b, j: (b, j, 0)),
            pl.BlockSpec(rc.shape, lambda b, j: (0, 0)),
            pl.BlockSpec((_IN, _HID), lambda b, j: (0, 0)),
            pl.BlockSpec((1, _HID), lambda b, j: (0, 0)),
            pl.BlockSpec((_HID, _EMB), lambda b, j: (0, 0)),
            pl.BlockSpec((1, _EMB), lambda b, j: (0, 0)),
        ],
        out_specs=pl.BlockSpec((1, _N_NODES, _FCHUNK, _EMB),
                               lambda b, j: (b, 0, j, 0)),
        out_shape=jax.ShapeDtypeStruct((B, _N_NODES, n, _EMB), jnp.float32),
        scratch_shapes=[
            pltpu.VMEM((_IN * _N_NODES, _N_NODES), jnp.float32),
            pltpu.VMEM((_IN * _N_NODES, _N_NODES), jnp.float32),
        ],
    )(x, rc, W_gcn, b_gcn.reshape(1, _HID), W_proj, b_proj.reshape(1, _EMB))

    return jnp.transpose(out_nm, (0, 2, 1, 3))


# speedup vs baseline: 28.7373x; 1.1035x over previous
import functools

import jax
import jax.numpy as jnp
from jax.experimental import pallas as pl
from jax.experimental.pallas import tpu as pltpu

_N_NODES = 102
_IN = 2
_HID = 512
_EMB = 256
_FCHUNK = 128


def _fused_kernel(x_ref, rc_ref, wg_ref, bg_ref, wp_ref, bp_ref, o_ref,
                  me_ref, mo_ref):
    fc = x_ref.shape[1]

    @pl.when((pl.program_id(0) == 0) & (pl.program_id(1) == 0))
    def _build_adjacency():
        rc = rc_ref[...]
        row = rc[0:1, :]
        col = rc[1:2, :]
        e_tot = rc.shape[1]
        iota_n = jax.lax.broadcasted_iota(jnp.int32, (_N_NODES, e_tot), 0)
        cmask = (iota_n == col).astype(jnp.float32)
        rmask = (iota_n == row).astype(jnp.float32)
        deg = jnp.sum(cmask, axis=1, keepdims=True)
        dinv = jnp.where(deg > 0, jax.lax.rsqrt(deg), 0.0)
        dinv_col = jnp.sum(cmask * dinv, axis=0, keepdims=True)
        dinv_row = jnp.sum(rmask * dinv, axis=0, keepdims=True)
        norm = dinv_row * dinv_col
        iota_k = jax.lax.broadcasted_iota(jnp.int32, (2 * _N_NODES, e_tot), 0)
        pe = (iota_k == 2 * row).astype(jnp.float32)
        po = (iota_k == 2 * row + 1).astype(jnp.float32)
        q = cmask * norm
        dims = (((1,), (1,)), ((), ()))
        me_ref[...] = jax.lax.dot_general(pe, q, dims,
                                          preferred_element_type=jnp.float32)
        mo_ref[...] = jax.lax.dot_general(po, q, dims,
                                          preferred_element_type=jnp.float32)

    x = x_ref[0]
    cdims = (((0,), (1,)), ((), ()))
    yet = jax.lax.dot_general(me_ref[...], x, cdims,
                              preferred_element_type=jnp.float32)
    yot = jax.lax.dot_general(mo_ref[...], x, cdims,
                              preferred_element_type=jnp.float32)
    wg = wg_ref[...]
    h = (yet[:, :, None] * wg[0][None, None, :]
         + yot[:, :, None] * wg[1][None, None, :]
         + bg_ref[...][None])
    h = jnp.maximum(h, 0.0).reshape(_N_NODES * fc, _HID)
    o2 = (jnp.dot(h, wp_ref[...], preferred_element_type=jnp.float32)
          + bp_ref[...])
    o_ref[0] = o2.reshape(_N_NODES, fc, _EMB)


@functools.partial(jax.jit, static_argnames=())
def kernel(x, edge_index, W_gcn, b_gcn, W_proj, b_proj):
    B, n, _ = x.shape

    loops = jnp.arange(_N_NODES, dtype=edge_index.dtype)
    rc = jnp.concatenate(
        [edge_index, jnp.stack([loops, loops], axis=0)], axis=1)

    grid = (B, n // _FCHUNK)

    out_nm = pl.pallas_call(
        _fused_kernel,
        grid=grid,
        in_specs=[
            pl.BlockSpec((1, _FCHUNK, _IN * _N_NODES), lambda b, j: (b, j, 0)),
            pl.BlockSpec(rc.shape, lambda b, j: (0, 0)),
            pl.BlockSpec((_IN, _HID), lambda b, j: (0, 0)),
            pl.BlockSpec((1, _HID), lambda b, j: (0, 0)),
            pl.BlockSpec((_HID, _EMB), lambda b, j: (0, 0)),
            pl.BlockSpec((1, _EMB), lambda b, j: (0, 0)),
        ],
        out_specs=pl.BlockSpec((1, _N_NODES, _FCHUNK, _EMB),
                               lambda b, j: (b, 0, j, 0)),
        out_shape=jax.ShapeDtypeStruct((B, _N_NODES, n, _EMB), jnp.float32),
        scratch_shapes=[
            pltpu.VMEM((_IN * _N_NODES, _N_NODES), jnp.float32),
            pltpu.VMEM((_IN * _N_NODES, _N_NODES), jnp.float32),
        ],
    )(x, rc, W_gcn, b_gcn.reshape(1, _HID), W_proj, b_proj.reshape(1, _EMB))

    return jnp.transpose(out_nm, (0, 2, 1, 3))
